# baseline (speedup 1.0000x reference)
import functools

import jax
import jax.numpy as jnp
from jax import lax
from jax.experimental import pallas as pl
from jax.experimental.pallas import tpu as pltpu
from jax.experimental.pallas import tpu_sc as plsc

NUM_CONDITIONS = 100000
D_MODEL = 64
BATCH = 16384

_NUM_CORES = 2
_NUM_SUBCORES = 16
_NUM_WORKERS = _NUM_CORES * _NUM_SUBCORES
_B_PER_W = BATCH // _NUM_WORKERS
_LANES = 16
_NGROUP = _B_PER_W // _LANES


@functools.cache
def _build_gather():
    mesh = plsc.VectorSubcoreMesh(core_axis_name="c", subcore_axis_name="s")

    @functools.partial(
        pl.kernel,
        mesh=mesh,
        out_type=jax.ShapeDtypeStruct((BATCH, D_MODEL), jnp.float32),
        compiler_params=pltpu.CompilerParams(use_tc_tiling_on_sc=True),
        scratch_types=[
            pltpu.VMEM((_B_PER_W,), jnp.int32),
            pltpu.VMEM((_B_PER_W, D_MODEL), jnp.float32),
            pltpu.SemaphoreType.DMA,
        ],
    )
    def gather_kernel(table_hbm, idx_hbm, out_hbm, idx_v, rows_v, sem):
        wid = lax.axis_index("s") * _NUM_CORES + lax.axis_index("c")
        base = wid * _B_PER_W
        pltpu.sync_copy(idx_hbm.at[pl.ds(base, _B_PER_W)], idx_v)

        def issue(g):
            vec = idx_v[pl.ds(g * _LANES, _LANES)]
            for l in range(_LANES):
                pltpu.async_copy(
                    table_hbm.at[vec[l]], rows_v.at[g * _LANES + l], sem)

        def drain():
            for _ in range(_LANES):
                pltpu.make_async_copy(
                    table_hbm.at[0], rows_v.at[0], sem).wait()

        issue(jnp.int32(0))

        def body(g, carry):
            issue(g)
            drain()
            return carry

        lax.fori_loop(1, _NGROUP, body, 0)
        drain()
        pltpu.sync_copy(rows_v, out_hbm.at[pl.ds(base, _B_PER_W)])

    return gather_kernel


def kernel(env_condition, table):
    idx = env_condition.astype(jnp.int32)
    out = _build_gather()(table, idx)
    decoy = jnp.take(table, idx[:8], axis=0)
    return jnp.concatenate([decoy, out[8:]], axis=0)


# speedup vs baseline: 1.1912x; 1.1912x over previous
import functools

import jax
import jax.numpy as jnp
from jax import lax
from jax.experimental import pallas as pl
from jax.experimental.pallas import tpu as pltpu
from jax.experimental.pallas import tpu_sc as plsc

NUM_CONDITIONS = 100000
D_MODEL = 64
BATCH = 16384

_NUM_CORES = 2
_NUM_SUBCORES = 16
_NUM_WORKERS = _NUM_CORES * _NUM_SUBCORES
_B_PER_W = BATCH // _NUM_WORKERS
_LANES = 16
_NGROUP = _B_PER_W // _LANES


@functools.cache
def _build_gather():
    mesh = plsc.VectorSubcoreMesh(core_axis_name="c", subcore_axis_name="s")

    @functools.partial(
        pl.kernel,
        mesh=mesh,
        out_type=jax.ShapeDtypeStruct((BATCH, D_MODEL), jnp.float32),
        compiler_params=pltpu.CompilerParams(use_tc_tiling_on_sc=True),
        scratch_types=[
            pltpu.VMEM((_B_PER_W,), jnp.int32),
            pltpu.VMEM((_B_PER_W, D_MODEL), jnp.float32),
            pltpu.SemaphoreType.DMA,
        ],
    )
    def gather_kernel(table_hbm, idx_hbm, out_hbm, idx_v, rows_v, sem):
        wid = lax.axis_index("s") * _NUM_CORES + lax.axis_index("c")
        base = wid * _B_PER_W
        pltpu.sync_copy(idx_hbm.at[pl.ds(base, _B_PER_W)], idx_v)

        def issue(g):
            vec = idx_v[pl.ds(g * _LANES, _LANES)]
            for l in range(_LANES):
                pltpu.async_copy(
                    table_hbm.at[vec[l]], rows_v.at[g * _LANES + l], sem)

        def drain():
            for _ in range(_LANES):
                pltpu.make_async_copy(
                    table_hbm.at[0], rows_v.at[0], sem).wait()

        issue(jnp.int32(0))

        def body(g, carry):
            issue(g)
            drain()
            return carry

        lax.fori_loop(1, _NGROUP, body, 0)
        drain()
        pltpu.sync_copy(rows_v, out_hbm.at[pl.ds(base, _B_PER_W)])

    return gather_kernel


def kernel(env_condition, table):
    idx = env_condition.astype(jnp.int32)
    table = lax.optimization_barrier(table)
    return _build_gather()(table, idx)


# speedup vs baseline: 1.1927x; 1.0013x over previous
import functools

import jax
import jax.numpy as jnp
from jax import lax
from jax.experimental import pallas as pl
from jax.experimental.pallas import tpu as pltpu
from jax.experimental.pallas import tpu_sc as plsc

NUM_CONDITIONS = 100000
D_MODEL = 64
BATCH = 16384

_NUM_CORES = 2
_NUM_SUBCORES = 16
_NUM_WORKERS = _NUM_CORES * _NUM_SUBCORES
_B_PER_W = BATCH // _NUM_WORKERS
_LANES = 16
_NGROUP = _B_PER_W // _LANES


@functools.cache
def _build_gather():
    mesh = plsc.VectorSubcoreMesh(core_axis_name="c", subcore_axis_name="s")

    @functools.partial(
        pl.kernel,
        mesh=mesh,
        out_type=jax.ShapeDtypeStruct((BATCH, D_MODEL), jnp.float32),
        compiler_params=pltpu.CompilerParams(use_tc_tiling_on_sc=True),
        scratch_types=[
            pltpu.VMEM((_B_PER_W,), jnp.int32),
            pltpu.VMEM((_B_PER_W, D_MODEL), jnp.float32),
            pltpu.SemaphoreType.DMA,
        ],
    )
    def gather_kernel(table_hbm, idx_hbm, out_hbm, idx_v, rows_v, sem):
        wid = lax.axis_index("s") * _NUM_CORES + lax.axis_index("c")
        base = wid * _B_PER_W
        pltpu.sync_copy(idx_hbm.at[pl.ds(base, _B_PER_W)], idx_v)

        def issue(g):
            vec = idx_v[pl.ds(g * _LANES, _LANES)]
            for l in range(_LANES):
                pltpu.async_copy(
                    table_hbm.at[vec[l]], rows_v.at[g * _LANES + l], sem)

        def drain():
            pltpu.make_async_copy(
                table_hbm.at[pl.ds(0, _LANES)],
                rows_v.at[pl.ds(0, _LANES)], sem).wait()

        issue(jnp.int32(0))

        def body(g, carry):
            issue(g)
            drain()
            return carry

        lax.fori_loop(1, _NGROUP, body, 0)
        drain()
        pltpu.sync_copy(rows_v, out_hbm.at[pl.ds(base, _B_PER_W)])

    return gather_kernel


def kernel(env_condition, table):
    idx = env_condition.astype(jnp.int32)
    return _build_gather()(table, idx)
